# baseline (speedup 1.0000x reference)
import functools

import jax
import jax.numpy as jnp
from jax import lax
from jax.experimental import pallas as pl
from jax.experimental.pallas import tpu as pltpu
from jax.experimental.pallas import tpu_sc as plsc

_SEQ = 4096
_BATCH = 16
_NB = 4
_HALF = _SEQ // 2
_LANES = 16


def _span_ends(u, sml, lrg, su):
    sc = jnp.where(u < jnp.float32(0.5), sml, lrg)
    ln = jnp.maximum((sc * jnp.float32(_SEQ)).astype(jnp.int32), 1)
    mx = jnp.maximum(_SEQ - ln, 0)
    st = (su * (mx.astype(jnp.float32) + jnp.float32(1.0))).astype(jnp.int32)
    en = jnp.minimum(st + ln, _SEQ)
    return st, en


def _sc_body(params_hbm, pos_out, params_v, pbuf, sem):
    c = lax.axis_index("c")
    s = lax.axis_index("s")
    row = s
    col0 = c * _HALF

    pltpu.async_copy(params_hbm, params_v, sem).wait()

    lanes = lax.iota(jnp.int32, _LANES)
    lo2 = lanes & (_NB - 1)

    def grab(q):
        return plsc.load_gather(
            params_v,
            [jnp.broadcast_to(q, (16,)), jnp.broadcast_to(row, (16,)), lo2])

    st, en = _span_ends(grab(0), grab(1), grab(2), grab(3))

    key = jnp.where(lanes < _NB, st, jnp.int32(2 ** 31 - 1))
    ks, vs = plsc.sort_key_val(key, en)
    ss = [ks[i] for i in range(_NB)]
    ee = [vs[i] for i in range(_NB)]

    a0, b0 = ss[0], ee[0]
    a1 = jnp.maximum(ss[1], b0); b1 = jnp.maximum(ee[1], b0)
    a2 = jnp.maximum(ss[2], b1); b2 = jnp.maximum(ee[2], b1)
    a3 = jnp.maximum(ss[3], b2); b3 = jnp.maximum(ee[3], b2)
    c1 = b0 - a0
    c2 = c1 + (b1 - a1)
    c3 = c2 + (b2 - a2)
    c4 = c3 + (b3 - a3)
    g1 = a1 - b0
    g2 = a2 - b1
    g3 = a3 - b2

    _UNROLL = 4

    def chunk(q, carry):
        base = q * (_LANES * _UNROLL)
        for r in range(_UNROLL):
            j = col0 + base + r * _LANES + lanes
            v = j + a0
            v = v + jnp.where(j >= c1, g1, 0)
            v = v + jnp.where(j >= c2, g2, 0)
            v = v + jnp.where(j >= c3, g3, 0)
            v = jnp.where(j >= c4, _SEQ, v)
            pbuf[pl.ds(base + r * _LANES, _LANES)] = v
        return carry

    lax.fori_loop(0, _HALF // (_LANES * _UNROLL), chunk, 0)

    pltpu.sync_copy(pbuf, pos_out.at[row, pl.ds(col0, _HALF)])


_sc_positions = pl.kernel(
    _sc_body,
    out_type=jax.ShapeDtypeStruct((_BATCH, _SEQ), jnp.int32),
    mesh=plsc.VectorSubcoreMesh(core_axis_name="c", subcore_axis_name="s"),
    scratch_types=[
        pltpu.VMEM((_NB, _BATCH, _NB), jnp.float32),
        pltpu.VMEM((_HALF,), jnp.int32),
        pltpu.SemaphoreType.DMA,
    ],
    compiler_params=pltpu.CompilerParams(needs_layout_passes=False),
    name="span_positions_sc",
)


def _tc_body(params_ref, cmask_ref, tmask_ref):
    st, en = _span_ends(params_ref[0], params_ref[1], params_ref[2],
                        params_ref[3])
    st16 = st.astype(jnp.int16)
    en16 = en.astype(jnp.int16)
    pos = lax.broadcasted_iota(jnp.int16, (_BATCH, _SEQ), 1)
    m = None
    for k in range(_NB):
        term = (pos >= st16[:, k:k + 1]) & (pos < en16[:, k:k + 1])
        m = term if m is None else m | term
    tmask_ref[...] = m.astype(jnp.int8)
    cmask_ref[...] = jnp.logical_not(m).astype(jnp.int8)


_tc_masks = pl.pallas_call(
    _tc_body,
    out_shape=[
        jax.ShapeDtypeStruct((_BATCH, _SEQ), jnp.int8),
        jax.ShapeDtypeStruct((_BATCH, _SEQ), jnp.int8),
    ],
    name="span_masks_tc",
)


def kernel(use_small_u, small_scales, large_scales, start_u):
    args = (use_small_u, small_scales, large_scales, start_u)
    params = jnp.stack([a.reshape(_BATCH, _NB) for a in args])
    positions = _sc_positions(params)
    cmask, tmask = _tc_masks(params)
    return (cmask.astype(jnp.bool_), tmask.astype(jnp.bool_), positions)


# speedup vs baseline: 1.0406x; 1.0406x over previous
import functools

import jax
import jax.numpy as jnp
from jax import lax
from jax.experimental import pallas as pl
from jax.experimental.pallas import tpu as pltpu
from jax.experimental.pallas import tpu_sc as plsc

_SEQ = 4096
_BATCH = 16
_NB = 4
_HALF = _SEQ // 2
_LANES = 16


def _span_ends(u, sml, lrg, su):
    sc = jnp.where(u < jnp.float32(0.5), sml, lrg)
    ln = jnp.maximum((sc * jnp.float32(_SEQ)).astype(jnp.int32), 1)
    mx = jnp.maximum(_SEQ - ln, 0)
    st = (su * (mx.astype(jnp.float32) + jnp.float32(1.0))).astype(jnp.int32)
    en = jnp.minimum(st + ln, _SEQ)
    return st, en


def _sc_body(use_hbm, small_hbm, large_hbm, start_hbm, pos_out,
             params_v, pbuf, sem):
    c = lax.axis_index("c")
    s = lax.axis_index("s")
    row = s
    col0 = c * _HALF

    srcs = (use_hbm, small_hbm, large_hbm, start_hbm)
    copies = [pltpu.async_copy(src, params_v.at[q], sem)
              for q, src in enumerate(srcs)]
    for cp in copies:
        cp.wait()

    lanes = lax.iota(jnp.int32, _LANES)
    gidx = row * _NB + (lanes & (_NB - 1))

    def grab(q):
        return plsc.load_gather(params_v, [jnp.broadcast_to(q, (16,)), gidx])

    st, en = _span_ends(grab(0), grab(1), grab(2), grab(3))

    key = jnp.where(lanes < _NB, st, jnp.int32(2 ** 31 - 1))
    ks, vs = plsc.sort_key_val(key, en)
    ss = [ks[i] for i in range(_NB)]
    ee = [vs[i] for i in range(_NB)]

    a0, b0 = ss[0], ee[0]
    a1 = jnp.maximum(ss[1], b0); b1 = jnp.maximum(ee[1], b0)
    a2 = jnp.maximum(ss[2], b1); b2 = jnp.maximum(ee[2], b1)
    a3 = jnp.maximum(ss[3], b2); b3 = jnp.maximum(ee[3], b2)
    c1 = b0 - a0
    c2 = c1 + (b1 - a1)
    c3 = c2 + (b2 - a2)
    c4 = c3 + (b3 - a3)
    g1 = a1 - b0
    g2 = a2 - b1
    g3 = a3 - b2

    _UNROLL = 2

    def chunk(q, carry):
        base = q * (_LANES * _UNROLL)
        for r in range(_UNROLL):
            j = col0 + base + r * _LANES + lanes
            v = j + a0
            v = v + jnp.where(j >= c1, g1, 0)
            v = v + jnp.where(j >= c2, g2, 0)
            v = v + jnp.where(j >= c3, g3, 0)
            v = jnp.where(j >= c4, _SEQ, v)
            pbuf[pl.ds(base + r * _LANES, _LANES)] = v
        return carry

    lax.fori_loop(0, _HALF // (_LANES * _UNROLL), chunk, 0)

    pltpu.sync_copy(pbuf, pos_out.at[row, pl.ds(col0, _HALF)])


_sc_positions = pl.kernel(
    _sc_body,
    out_type=jax.ShapeDtypeStruct((_BATCH, _SEQ), jnp.int32),
    mesh=plsc.VectorSubcoreMesh(core_axis_name="c", subcore_axis_name="s"),
    scratch_types=[
        pltpu.VMEM((_NB, _BATCH * _NB), jnp.float32),
        pltpu.VMEM((_HALF,), jnp.int32),
        pltpu.SemaphoreType.DMA,
    ],
    compiler_params=pltpu.CompilerParams(needs_layout_passes=False),
    name="span_positions_sc",
)


def _tc_body(params_ref, cmask_ref, tmask_ref):
    st, en = _span_ends(params_ref[0], params_ref[1], params_ref[2],
                        params_ref[3])
    st16 = st.astype(jnp.int16)
    en16 = en.astype(jnp.int16)
    pos = lax.broadcasted_iota(jnp.int16, (_BATCH, _SEQ), 1)
    m = None
    for k in range(_NB):
        term = (pos >= st16[:, k:k + 1]) & (pos < en16[:, k:k + 1])
        m = term if m is None else m | term
    tmask_ref[...] = m.astype(jnp.int8)
    cmask_ref[...] = jnp.logical_not(m).astype(jnp.int8)


_tc_masks = pl.pallas_call(
    _tc_body,
    out_shape=[
        jax.ShapeDtypeStruct((_BATCH, _SEQ), jnp.int8),
        jax.ShapeDtypeStruct((_BATCH, _SEQ), jnp.int8),
    ],
    name="span_masks_tc",
)


def kernel(use_small_u, small_scales, large_scales, start_u):
    args = (use_small_u, small_scales, large_scales, start_u)
    positions = _sc_positions(*args)
    params = jnp.stack([a.reshape(_BATCH, _NB) for a in args])
    cmask, tmask = _tc_masks(params)
    return (cmask.astype(jnp.bool_), tmask.astype(jnp.bool_), positions)


# speedup vs baseline: 1.1079x; 1.0647x over previous
import functools

import jax
import jax.numpy as jnp
from jax import lax
from jax.experimental import pallas as pl
from jax.experimental.pallas import tpu as pltpu
from jax.experimental.pallas import tpu_sc as plsc

_SEQ = 4096
_BATCH = 16
_NB = 4
_NCORES = 1
_WIDTH = _SEQ // _NCORES
_LANES = 16


def _span_ends(u, sml, lrg, su):
    sc = jnp.where(u < jnp.float32(0.5), sml, lrg)
    ln = jnp.maximum((sc * jnp.float32(_SEQ)).astype(jnp.int32), 1)
    mx = jnp.maximum(_SEQ - ln, 0)
    st = (su * (mx.astype(jnp.float32) + jnp.float32(1.0))).astype(jnp.int32)
    en = jnp.minimum(st + ln, _SEQ)
    return st, en


def _sc_body(use_hbm, small_hbm, large_hbm, start_hbm, pos_out,
             params_v, pbuf, sem):
    c = lax.axis_index("c")
    s = lax.axis_index("s")
    row = s
    col0 = c * _WIDTH

    srcs = (use_hbm, small_hbm, large_hbm, start_hbm)
    copies = [pltpu.async_copy(src, params_v.at[q], sem)
              for q, src in enumerate(srcs)]
    for cp in copies:
        cp.wait()

    lanes = lax.iota(jnp.int32, _LANES)
    gidx = row * _NB + (lanes & (_NB - 1))

    def grab(q):
        return plsc.load_gather(params_v, [jnp.broadcast_to(q, (16,)), gidx])

    st, en = _span_ends(grab(0), grab(1), grab(2), grab(3))

    key = jnp.where(lanes < _NB, st, jnp.int32(2 ** 31 - 1))
    ks, vs = plsc.sort_key_val(key, en)
    ss = [ks[i] for i in range(_NB)]
    ee = [vs[i] for i in range(_NB)]

    a0, b0 = ss[0], ee[0]
    a1 = jnp.maximum(ss[1], b0); b1 = jnp.maximum(ee[1], b0)
    a2 = jnp.maximum(ss[2], b1); b2 = jnp.maximum(ee[2], b1)
    a3 = jnp.maximum(ss[3], b2); b3 = jnp.maximum(ee[3], b2)
    c1 = b0 - a0
    c2 = c1 + (b1 - a1)
    c3 = c2 + (b2 - a2)
    c4 = c3 + (b3 - a3)
    g1 = a1 - b0
    g2 = a2 - b1
    g3 = a3 - b2

    _UNROLL = 2

    def chunk(q, carry):
        base = q * (_LANES * _UNROLL)
        for r in range(_UNROLL):
            j = col0 + base + r * _LANES + lanes
            v = j + a0
            v = v + jnp.where(j >= c1, g1, 0)
            v = v + jnp.where(j >= c2, g2, 0)
            v = v + jnp.where(j >= c3, g3, 0)
            v = jnp.where(j >= c4, _SEQ, v)
            pbuf[pl.ds(base + r * _LANES, _LANES)] = v
        return carry

    lax.fori_loop(0, _WIDTH // (_LANES * _UNROLL), chunk, 0)

    pltpu.sync_copy(pbuf, pos_out.at[row, pl.ds(col0, _WIDTH)])


_sc_positions = pl.kernel(
    _sc_body,
    out_type=jax.ShapeDtypeStruct((_BATCH, _SEQ), jnp.int32),
    mesh=plsc.VectorSubcoreMesh(core_axis_name="c", subcore_axis_name="s",
                                num_cores=_NCORES),
    scratch_types=[
        pltpu.VMEM((_NB, _BATCH * _NB), jnp.float32),
        pltpu.VMEM((_WIDTH,), jnp.int32),
        pltpu.SemaphoreType.DMA,
    ],
    compiler_params=pltpu.CompilerParams(needs_layout_passes=False),
    name="span_positions_sc",
)


def _tc_body(params_ref, cmask_ref, tmask_ref):
    st, en = _span_ends(params_ref[0], params_ref[1], params_ref[2],
                        params_ref[3])
    st16 = st.astype(jnp.int16)
    en16 = en.astype(jnp.int16)
    pos = lax.broadcasted_iota(jnp.int16, (_BATCH, _SEQ), 1)
    m = None
    for k in range(_NB):
        term = (pos >= st16[:, k:k + 1]) & (pos < en16[:, k:k + 1])
        m = term if m is None else m | term
    tmask_ref[...] = m.astype(jnp.int8)
    cmask_ref[...] = jnp.logical_not(m).astype(jnp.int8)


_tc_masks = pl.pallas_call(
    _tc_body,
    out_shape=[
        jax.ShapeDtypeStruct((_BATCH, _SEQ), jnp.int8),
        jax.ShapeDtypeStruct((_BATCH, _SEQ), jnp.int8),
    ],
    name="span_masks_tc",
)


def kernel(use_small_u, small_scales, large_scales, start_u):
    args = (use_small_u, small_scales, large_scales, start_u)
    positions = _sc_positions(*args)
    params = jnp.stack([a.reshape(_BATCH, _NB) for a in args])
    cmask, tmask = _tc_masks(params)
    return (cmask.astype(jnp.bool_), tmask.astype(jnp.bool_), positions)


# speedup vs baseline: 1.1121x; 1.0038x over previous
import functools

import jax
import jax.numpy as jnp
from jax import lax
from jax.experimental import pallas as pl
from jax.experimental.pallas import tpu as pltpu
from jax.experimental.pallas import tpu_sc as plsc

_SEQ = 4096
_BATCH = 16
_NB = 4
_NCORES = 1
_WIDTH = _SEQ // _NCORES
_LANES = 16


def _span_ends(u, sml, lrg, su):
    sc = jnp.where(u < jnp.float32(0.5), sml, lrg)
    ln = jnp.maximum((sc * jnp.float32(_SEQ)).astype(jnp.int32), 1)
    mx = jnp.maximum(_SEQ - ln, 0)
    st = (su * (mx.astype(jnp.float32) + jnp.float32(1.0))).astype(jnp.int32)
    en = jnp.minimum(st + ln, _SEQ)
    return st, en


def _sc_body(use_hbm, small_hbm, large_hbm, start_hbm, pos_out,
             params_v, pbuf, sem):
    c = lax.axis_index("c")
    s = lax.axis_index("s")
    row = s
    col0 = c * _WIDTH

    srcs = (use_hbm, small_hbm, large_hbm, start_hbm)
    copies = [pltpu.async_copy(src, params_v.at[q], sem)
              for q, src in enumerate(srcs)]
    for cp in copies:
        cp.wait()

    lanes = lax.iota(jnp.int32, _LANES)
    gidx = row * _NB + (lanes & (_NB - 1))

    def grab(q):
        return plsc.load_gather(params_v, [jnp.broadcast_to(q, (16,)), gidx])

    st, en = _span_ends(grab(0), grab(1), grab(2), grab(3))

    key = jnp.where(lanes < _NB, st, jnp.int32(2 ** 31 - 1))
    ks, vs = plsc.sort_key_val(key, en)
    ss = [ks[i] for i in range(_NB)]
    ee = [vs[i] for i in range(_NB)]

    a0, b0 = ss[0], ee[0]
    a1 = jnp.maximum(ss[1], b0); b1 = jnp.maximum(ee[1], b0)
    a2 = jnp.maximum(ss[2], b1); b2 = jnp.maximum(ee[2], b1)
    a3 = jnp.maximum(ss[3], b2); b3 = jnp.maximum(ee[3], b2)
    c1 = b0 - a0
    c2 = c1 + (b1 - a1)
    c3 = c2 + (b2 - a2)
    c4 = c3 + (b3 - a3)
    g1 = a1 - b0
    g2 = a2 - b1
    g3 = a3 - b2

    _UNROLL = 2

    def chunk(q, carry):
        base = q * (_LANES * _UNROLL)
        for r in range(_UNROLL):
            j = col0 + base + r * _LANES + lanes
            v = j + a0
            v = v + jnp.where(j >= c1, g1, 0)
            v = v + jnp.where(j >= c2, g2, 0)
            v = v + jnp.where(j >= c3, g3, 0)
            v = jnp.where(j >= c4, _SEQ, v)
            pbuf[pl.ds(base + r * _LANES, _LANES)] = v
        return carry

    lax.fori_loop(0, _WIDTH // (_LANES * _UNROLL), chunk, 0)

    pltpu.sync_copy(pbuf, pos_out.at[row, pl.ds(col0, _WIDTH)])


_sc_positions = pl.kernel(
    _sc_body,
    out_type=jax.ShapeDtypeStruct((_BATCH, _SEQ), jnp.int32),
    mesh=plsc.VectorSubcoreMesh(core_axis_name="c", subcore_axis_name="s",
                                num_cores=_NCORES),
    scratch_types=[
        pltpu.VMEM((_NB, _BATCH * _NB), jnp.float32),
        pltpu.VMEM((_WIDTH,), jnp.int32),
        pltpu.SemaphoreType.DMA,
    ],
    compiler_params=pltpu.CompilerParams(needs_layout_passes=False,
                                        skip_device_barrier=True),
    name="span_positions_sc",
)


def _tc_body(params_ref, cmask_ref, tmask_ref):
    st, en = _span_ends(params_ref[0], params_ref[1], params_ref[2],
                        params_ref[3])
    st16 = st.astype(jnp.int16)
    en16 = en.astype(jnp.int16)
    pos = lax.broadcasted_iota(jnp.int16, (_BATCH, _SEQ), 1)
    m = None
    for k in range(_NB):
        term = (pos >= st16[:, k:k + 1]) & (pos < en16[:, k:k + 1])
        m = term if m is None else m | term
    tmask_ref[...] = m.astype(jnp.int8)
    cmask_ref[...] = jnp.logical_not(m).astype(jnp.int8)


_tc_masks = pl.pallas_call(
    _tc_body,
    out_shape=[
        jax.ShapeDtypeStruct((_BATCH, _SEQ), jnp.int8),
        jax.ShapeDtypeStruct((_BATCH, _SEQ), jnp.int8),
    ],
    name="span_masks_tc",
)


def kernel(use_small_u, small_scales, large_scales, start_u):
    args = (use_small_u, small_scales, large_scales, start_u)
    positions = _sc_positions(*args)
    params = jnp.stack([a.reshape(_BATCH, _NB) for a in args])
    cmask, tmask = _tc_masks(params)
    return (cmask.astype(jnp.bool_), tmask.astype(jnp.bool_), positions)
